# baseline (speedup 1.0000x reference)
import jax
import jax.numpy as jnp
from jax import lax
from jax.experimental import pallas as pl
from jax.experimental.pallas import tpu as pltpu
from jax.experimental.pallas import tpu_sc as plsc

NC = 2
NS = 16
L = 16
NW = NC * NS

N = 4096 * 64 * 64
PER_W = N // NW
CHUNK = 8192
NCHUNK = PER_W // CHUNK
GRP = 128

MAGIC = 8388608.0
RGB_SCALE = 127.5


def _sc_body(x_hbm, bk_hbm, mm_hbm, out_hbm, bk_v, mm_v, rtab_v, gtab_v,
             btab_v, in_bufs, out_bufs, isems, osems):
    wid = lax.axis_index("s") * NC + lax.axis_index("c")
    base = wid * PER_W

    pltpu.sync_copy(bk_hbm, bk_v)
    pltpu.sync_copy(mm_hbm, mm_v)
    for j in range(32 // L):
        bk = jnp.clip(bk_v[pl.ds(j * L, L)], 0, 31)
        mv = plsc.load_gather(mm_v, [bk])
        r = jnp.bitwise_and(jnp.right_shift(mv, 16), 255).astype(jnp.float32)
        g = jnp.bitwise_and(jnp.right_shift(mv, 8), 255).astype(jnp.float32)
        b = jnp.bitwise_and(mv, 255).astype(jnp.float32)
        rtab_v[pl.ds(j * L, L)] = (r - RGB_SCALE) / RGB_SCALE
        gtab_v[pl.ds(j * L, L)] = (g - RGB_SCALE) / RGB_SCALE
        btab_v[pl.ds(j * L, L)] = (b - RGB_SCALE) / RGB_SCALE

    for b in range(2):
        pltpu.async_copy(x_hbm.at[pl.ds(base + b * CHUNK, CHUNK)], in_bufs[b],
                         isems[b])

    def process(c, in_v, out_v, isem, osem):
        pltpu.make_async_copy(x_hbm.at[pl.ds(0, CHUNK)], in_v, isem).wait()

        @pl.when(c >= 2)
        def _wait_out():
            pltpu.make_async_copy(out_v, out_hbm.at[pl.ds(0, 4 * CHUNK)],
                                  osem).wait()

        @plsc.parallel_loop(0, CHUNK // GRP, step=1, unroll=4)
        def _grp(j):
            ib = j * GRP
            ob = j * (4 * GRP)
            for v in range(GRP // L):
                xv = in_v[pl.ds(ib + v * L, L)]
                t = ((xv + 1.0) * 0.5) * 31.0
                idx = jnp.clip(((t + MAGIC) - MAGIC).astype(jnp.int32), 0, 31)
                out_v[pl.ds(ob + v * L, L)] = xv
                out_v[pl.ds(ob + GRP + v * L, L)] = plsc.load_gather(
                    rtab_v, [idx])
                out_v[pl.ds(ob + 2 * GRP + v * L, L)] = plsc.load_gather(
                    gtab_v, [idx])
                out_v[pl.ds(ob + 3 * GRP + v * L, L)] = plsc.load_gather(
                    btab_v, [idx])

        off = base + c * CHUNK
        pltpu.async_copy(out_v, out_hbm.at[pl.ds(off * 4, 4 * CHUNK)], osem)

        @pl.when(c + 2 < NCHUNK)
        def _prefetch():
            pltpu.async_copy(x_hbm.at[pl.ds(off + 2 * CHUNK, CHUNK)], in_v,
                             isem)

    def chunk_pair(k, carry):
        process(2 * k, in_bufs[0], out_bufs[0], isems[0], osems[0])
        process(2 * k + 1, in_bufs[1], out_bufs[1], isems[1], osems[1])
        return carry

    lax.fori_loop(0, NCHUNK // 2, chunk_pair, 0)

    for b in range(2):
        pltpu.make_async_copy(out_bufs[b], out_hbm.at[pl.ds(0, 4 * CHUNK)],
                              osems[b]).wait()


@jax.jit
def kernel(x, backwards_table, minimap_table):
    x_flat = jnp.transpose(x, (1, 2, 3, 0)).reshape(N)
    run = pl.kernel(
        _sc_body,
        out_type=jax.ShapeDtypeStruct((4 * N,), jnp.float32),
        mesh=plsc.VectorSubcoreMesh(core_axis_name="c", subcore_axis_name="s"),
        compiler_params=pltpu.CompilerParams(needs_layout_passes=False),
        scratch_types=[
            pltpu.VMEM((32,), jnp.int32),
            pltpu.VMEM((32,), jnp.int32),
            pltpu.VMEM((32,), jnp.float32),
            pltpu.VMEM((32,), jnp.float32),
            pltpu.VMEM((32,), jnp.float32),
            [pltpu.VMEM((CHUNK,), jnp.float32) for _ in range(2)],
            [pltpu.VMEM((4 * CHUNK,), jnp.float32) for _ in range(2)],
            [pltpu.SemaphoreType.DMA for _ in range(2)],
            [pltpu.SemaphoreType.DMA for _ in range(2)],
        ],
    )
    out_flat = run(x_flat, backwards_table.astype(jnp.int32),
                   minimap_table.astype(jnp.int32))
    z = out_flat.reshape(64, 64, 4096 // GRP, 4, GRP)
    return jnp.transpose(z, (2, 4, 0, 1, 3)).reshape(4096, 64, 64, 4)


# speedup vs baseline: 1.2482x; 1.2482x over previous
import jax
import jax.numpy as jnp
from jax import lax
from jax.experimental import pallas as pl
from jax.experimental.pallas import tpu as pltpu
from jax.experimental.pallas import tpu_sc as plsc

NC = 2
NS = 16
L = 16
NW = NC * NS

N = 4096 * 64 * 64
PER_W = N // NW
CHUNK = 8192
NCHUNK = PER_W // CHUNK
GRP = 128

MAGIC = 8388608.0
RGB_SCALE = 127.5


def _sc_body(x_hbm, bk_hbm, mm_hbm, out_hbm, bk_v, mm_v, rtab_v, gtab_v,
             btab_v, in_bufs, out_bufs, isems, osems):
    wid = lax.axis_index("s") * NC + lax.axis_index("c")
    base = wid * PER_W

    pltpu.sync_copy(bk_hbm, bk_v)
    pltpu.sync_copy(mm_hbm, mm_v)
    for j in range(32 // L):
        bk = jnp.clip(bk_v[pl.ds(j * L, L)], 0, 31)
        mv = plsc.load_gather(mm_v, [bk])
        r = jnp.bitwise_and(jnp.right_shift(mv, 16), 255).astype(jnp.float32)
        g = jnp.bitwise_and(jnp.right_shift(mv, 8), 255).astype(jnp.float32)
        b = jnp.bitwise_and(mv, 255).astype(jnp.float32)
        rtab_v[pl.ds(j * L, L)] = (r - RGB_SCALE) / RGB_SCALE
        gtab_v[pl.ds(j * L, L)] = (g - RGB_SCALE) / RGB_SCALE
        btab_v[pl.ds(j * L, L)] = (b - RGB_SCALE) / RGB_SCALE

    for b in range(2):
        pltpu.async_copy(x_hbm.at[pl.ds(base + b * CHUNK, CHUNK)], in_bufs[b],
                         isems[b])

    def process(c, in_v, out_v, isem, osem):
        pltpu.make_async_copy(x_hbm.at[pl.ds(0, CHUNK)], in_v, isem).wait()

        @pl.when(c >= 2)
        def _wait_out():
            pltpu.make_async_copy(out_v, out_hbm.at[pl.ds(0, 4 * CHUNK)],
                                  osem).wait()

        @plsc.parallel_loop(0, CHUNK // GRP, step=1, unroll=1)
        def _grp(j):
            ib = j * GRP
            ob = j * (4 * GRP)
            for v in range(GRP // L):
                xv = in_v[pl.ds(ib + v * L, L)]
                t = ((xv + 1.0) * 0.5) * 31.0
                idx = jnp.clip(((t + MAGIC) - MAGIC).astype(jnp.int32), 0, 31)
                out_v[pl.ds(ob + v * L, L)] = xv
                out_v[pl.ds(ob + GRP + v * L, L)] = plsc.load_gather(
                    rtab_v, [idx])
                out_v[pl.ds(ob + 2 * GRP + v * L, L)] = plsc.load_gather(
                    gtab_v, [idx])
                out_v[pl.ds(ob + 3 * GRP + v * L, L)] = plsc.load_gather(
                    btab_v, [idx])

        off = base + c * CHUNK
        pltpu.async_copy(out_v, out_hbm.at[pl.ds(off * 4, 4 * CHUNK)], osem)

        @pl.when(c + 2 < NCHUNK)
        def _prefetch():
            pltpu.async_copy(x_hbm.at[pl.ds(off + 2 * CHUNK, CHUNK)], in_v,
                             isem)

    def chunk_pair(k, carry):
        process(2 * k, in_bufs[0], out_bufs[0], isems[0], osems[0])
        process(2 * k + 1, in_bufs[1], out_bufs[1], isems[1], osems[1])
        return carry

    lax.fori_loop(0, NCHUNK // 2, chunk_pair, 0)

    for b in range(2):
        pltpu.make_async_copy(out_bufs[b], out_hbm.at[pl.ds(0, 4 * CHUNK)],
                              osems[b]).wait()


@jax.jit
def kernel(x, backwards_table, minimap_table):
    x_flat = jnp.transpose(x, (1, 2, 3, 0)).reshape(N)
    run = pl.kernel(
        _sc_body,
        out_type=jax.ShapeDtypeStruct((4 * N,), jnp.float32),
        mesh=plsc.VectorSubcoreMesh(core_axis_name="c", subcore_axis_name="s"),
        compiler_params=pltpu.CompilerParams(needs_layout_passes=False),
        scratch_types=[
            pltpu.VMEM((32,), jnp.int32),
            pltpu.VMEM((32,), jnp.int32),
            pltpu.VMEM((32,), jnp.float32),
            pltpu.VMEM((32,), jnp.float32),
            pltpu.VMEM((32,), jnp.float32),
            [pltpu.VMEM((CHUNK,), jnp.float32) for _ in range(2)],
            [pltpu.VMEM((4 * CHUNK,), jnp.float32) for _ in range(2)],
            [pltpu.SemaphoreType.DMA for _ in range(2)],
            [pltpu.SemaphoreType.DMA for _ in range(2)],
        ],
    )
    out_flat = run(x_flat, backwards_table.astype(jnp.int32),
                   minimap_table.astype(jnp.int32))
    z = out_flat.reshape(64, 64, 4096 // GRP, 4, GRP)
    return jnp.transpose(z, (2, 4, 0, 1, 3)).reshape(4096, 64, 64, 4)
